# Initial kernel scaffold; baseline (speedup 1.0000x reference)
import jax
import jax.numpy as jnp
from jax.experimental import pallas as pl


def kernel(v, f, attrs):
    raise NotImplementedError("write your pallas kernel here")


# speedup vs baseline: 28.6577x; 28.6577x over previous
import numpy as np

import jax
import jax.numpy as jnp
from jax import lax
from jax.experimental import pallas as pl
from jax.experimental.pallas import tpu as pltpu
from jax.experimental.pallas import tpu_sc as plsc

H = 512
W = 512
B = 4
NF = 100000
NTAB = B * NF
HWPIX = H * W
NP = B * HWPIX

NC, NS, L = 2, 16, 16
NW = NC * NS
PIX_PER_W = NP // NW
C = 1024
NG = C // L
IDXB = 128
NIDX = C // IDXB
NCHUNK = PIX_PER_W // C

_MUL = np.uint32(2654435761)


def _hash_pix(pvec_u32):
    hsh = pvec_u32 * _MUL
    tri = (hsh % np.uint32(NTAB)).astype(jnp.int32)
    valid = (hsh % np.uint32(7)) != np.uint32(0)
    validf = jnp.where(valid, np.float32(1.0), np.float32(0.0))
    b0 = ((hsh >> 3) % np.uint32(1024)).astype(jnp.float32) + 1.0
    b1 = ((hsh >> 13) % np.uint32(1024)).astype(jnp.float32) + 1.0
    b2 = ((hsh >> 23) % np.uint32(512)).astype(jnp.float32) + 1.0
    scale = validf / (b0 + b1 + b2)
    return tri, b0 * scale, b1 * scale, b2 * scale, validf


def _body(table, out, idxbuf, wbuf, gbuf, tbuf, sem):
    wid = lax.axis_index("s") * NC + lax.axis_index("c")
    pix0 = wid * PIX_PER_W
    b = pix0 // HWPIX
    pb0 = pix0 % HWPIX
    lane = lax.broadcasted_iota(jnp.int32, (16,), 0)
    lane_u = lane.astype(jnp.uint32)

    def chunk(c, _):
        pbase = pix0 + c * C

        def gen(g, _):
            pvec = (pbase + g * L).astype(jnp.uint32) + lane_u
            tri, w0, w1, w2, vis = _hash_pix(pvec)
            idxbuf[g // (IDXB // L), pl.ds((g % (IDXB // L)) * L, L)] = tri
            wbuf[0, pl.ds(g * L, L)] = w0
            wbuf[1, pl.ds(g * L, L)] = w1
            wbuf[2, pl.ds(g * L, L)] = w2
            tbuf[16, pl.ds(g * L, L)] = vis
            return 0

        lax.fori_loop(0, NG, gen, 0, unroll=4)

        copies = [
            pltpu.async_copy(
                table.at[idxbuf.at[j]], gbuf.at[pl.ds(j * IDXB, IDXB)], sem
            )
            for j in range(NIDX)
        ]
        for cp in copies:
            cp.wait()

        def wsum(g, _):
            rowv = g * L + lane
            w0 = wbuf[0, pl.ds(g * L, L)]
            w1 = wbuf[1, pl.ds(g * L, L)]
            w2 = wbuf[2, pl.ds(g * L, L)]
            for d in range(16):
                r0 = plsc.load_gather(gbuf, [rowv, lane * 0 + d])
                r1 = plsc.load_gather(gbuf, [rowv, lane * 0 + (16 + d)])
                r2 = plsc.load_gather(gbuf, [rowv, lane * 0 + (32 + d)])
                tbuf[d, pl.ds(g * L, L)] = w0 * r0 + w1 * r1 + w2 * r2
            return 0

        lax.fori_loop(0, NG, wsum, 0)

        pix_off = (b * 17) * HWPIX + pb0 + c * C
        for d in range(17):
            pltpu.sync_copy(tbuf.at[d], out.at[pl.ds(pix_off + d * HWPIX, C)])
        return 0

    lax.fori_loop(0, NCHUNK, chunk, 0)


@jax.jit
def _sc_render(table):
    mesh = plsc.VectorSubcoreMesh(core_axis_name="c", subcore_axis_name="s")
    return pl.kernel(
        _body,
        out_type=jax.ShapeDtypeStruct((B * 17 * HWPIX,), jnp.float32),
        mesh=mesh,
        scratch_types=[
            pltpu.VMEM((NIDX, IDXB), jnp.int32),
            pltpu.VMEM((3, C), jnp.float32),
            pltpu.VMEM((C, 48), jnp.float32),
            pltpu.VMEM((17, C), jnp.float32),
            pltpu.SemaphoreType.DMA,
        ],
        compiler_params=pltpu.CompilerParams(
            use_tc_tiling_on_sc=False, needs_layout_passes=False
        ),
    )(table)


def kernel(v, f, attrs):
    del v, f
    table = attrs.reshape(NTAB, 48)
    return _sc_render(table).reshape(B, 17, H, W)


# speedup vs baseline: 31.9356x; 1.1144x over previous
import numpy as np

import jax
import jax.numpy as jnp
from jax import lax
from jax.experimental import pallas as pl
from jax.experimental.pallas import tpu as pltpu
from jax.experimental.pallas import tpu_sc as plsc

H = 512
W = 512
B = 4
NF = 100000
NTAB = B * NF
HWPIX = H * W
NP = B * HWPIX

NC, NS, L = 2, 16, 16
NW = NC * NS
PIX_PER_W = NP // NW
C = 512
NG = C // L
IDXB = 128
NIDX = C // IDXB
NCHUNK = PIX_PER_W // C

_MUL = np.uint32(2654435761)


def _hash_pix(pvec_u32):
    hsh = pvec_u32 * _MUL
    tri = (hsh % np.uint32(NTAB)).astype(jnp.int32)
    valid = (hsh % np.uint32(7)) != np.uint32(0)
    validf = jnp.where(valid, np.float32(1.0), np.float32(0.0))
    b0 = ((hsh >> 3) % np.uint32(1024)).astype(jnp.float32) + 1.0
    b1 = ((hsh >> 13) % np.uint32(1024)).astype(jnp.float32) + 1.0
    b2 = ((hsh >> 23) % np.uint32(512)).astype(jnp.float32) + 1.0
    scale = validf / (b0 + b1 + b2)
    return tri, b0 * scale, b1 * scale, b2 * scale


def _body(table, out, idxA, idxB_, wA, wB, gA, gB, tA, tB, gsA, gsB, osA, osB):
    wid = lax.axis_index("s") * NC + lax.axis_index("c")
    pix0 = wid * PIX_PER_W
    b = pix0 // HWPIX
    out0 = (b * 17) * HWPIX + pix0 % HWPIX
    lane = lax.broadcasted_iota(jnp.int32, (16,), 0)
    lane_u = lane.astype(jnp.uint32)

    def gen(c, idxbuf, wbuf):
        pbase = pix0 + c * C

        def one(g, _):
            pvec = (pbase + g * L).astype(jnp.uint32) + lane_u
            tri, w0, w1, w2 = _hash_pix(pvec)
            idxbuf[g // (IDXB // L), pl.ds((g % (IDXB // L)) * L, L)] = tri
            wbuf[0, pl.ds(g * L, L)] = w0
            wbuf[1, pl.ds(g * L, L)] = w1
            wbuf[2, pl.ds(g * L, L)] = w2
            return 0

        lax.fori_loop(0, NG, one, 0, unroll=4)

    def fire_gather(idxbuf, gbuf, gsem):
        for j in range(NIDX):
            pltpu.async_copy(
                table.at[idxbuf.at[j]], gbuf.at[pl.ds(j * IDXB, IDXB)], gsem
            )

    def wait_gather(idxbuf, gbuf, gsem):
        for j in range(NIDX):
            pltpu.make_async_copy(
                table.at[idxbuf.at[j]], gbuf.at[pl.ds(j * IDXB, IDXB)], gsem
            ).wait()

    def wsum(gbuf, wbuf, tbuf):
        def one(g, _):
            rowv = g * L + lane
            w0 = wbuf[0, pl.ds(g * L, L)]
            w1 = wbuf[1, pl.ds(g * L, L)]
            w2 = wbuf[2, pl.ds(g * L, L)]
            tbuf[16, pl.ds(g * L, L)] = (w0 + w1) + w2
            for d in range(16):
                r0 = plsc.load_gather(gbuf, [rowv, lane * 0 + d])
                r1 = plsc.load_gather(gbuf, [rowv, lane * 0 + (16 + d)])
                r2 = plsc.load_gather(gbuf, [rowv, lane * 0 + (32 + d)])
                tbuf[d, pl.ds(g * L, L)] = w0 * r0 + w1 * r1 + w2 * r2
            return 0

        lax.fori_loop(0, NG, one, 0)

    def fire_out(c, tbuf, osem):
        off = out0 + c * C
        for d in range(17):
            pltpu.async_copy(tbuf.at[d], out.at[pl.ds(off + d * HWPIX, C)], osem)

    def drain_out(tbuf, osem):
        for d in range(17):
            pltpu.make_async_copy(tbuf.at[d], out.at[pl.ds(d * HWPIX, C)], osem).wait()

    gen(0, idxA, wA)
    fire_gather(idxA, gA, gsA)

    def step(t, _):
        c0 = 2 * t
        c1 = c0 + 1
        gen(c1, idxB_, wB)
        fire_gather(idxB_, gB, gsB)

        @pl.when(t > 0)
        def _():
            drain_out(tA, osA)

        wait_gather(idxA, gA, gsA)
        wsum(gA, wA, tA)
        fire_out(c0, tA, osA)

        @pl.when(c0 + 2 < NCHUNK)
        def _():
            gen(c0 + 2, idxA, wA)
            fire_gather(idxA, gA, gsA)

        @pl.when(t > 0)
        def _():
            drain_out(tB, osB)

        wait_gather(idxB_, gB, gsB)
        wsum(gB, wB, tB)
        fire_out(c1, tB, osB)
        return 0

    lax.fori_loop(0, NCHUNK // 2, step, 0)
    drain_out(tA, osA)
    drain_out(tB, osB)


@jax.jit
def _sc_render(table):
    mesh = plsc.VectorSubcoreMesh(core_axis_name="c", subcore_axis_name="s")
    return pl.kernel(
        _body,
        out_type=jax.ShapeDtypeStruct((B * 17 * HWPIX,), jnp.float32),
        mesh=mesh,
        scratch_types=[
            pltpu.VMEM((NIDX, IDXB), jnp.int32),
            pltpu.VMEM((NIDX, IDXB), jnp.int32),
            pltpu.VMEM((3, C), jnp.float32),
            pltpu.VMEM((3, C), jnp.float32),
            pltpu.VMEM((C, 48), jnp.float32),
            pltpu.VMEM((C, 48), jnp.float32),
            pltpu.VMEM((17, C), jnp.float32),
            pltpu.VMEM((17, C), jnp.float32),
            pltpu.SemaphoreType.DMA,
            pltpu.SemaphoreType.DMA,
            pltpu.SemaphoreType.DMA,
            pltpu.SemaphoreType.DMA,
        ],
        compiler_params=pltpu.CompilerParams(
            use_tc_tiling_on_sc=False, needs_layout_passes=False
        ),
    )(table)


def kernel(v, f, attrs):
    del v, f
    table = attrs.reshape(NTAB, 48)
    return _sc_render(table).reshape(B, 17, H, W)


# speedup vs baseline: 44.0729x; 1.3801x over previous
import numpy as np

import jax
import jax.numpy as jnp
from jax import lax
from jax.experimental import pallas as pl
from jax.experimental.pallas import tpu as pltpu
from jax.experimental.pallas import tpu_sc as plsc

H = 512
W = 512
B = 4
NF = 100000
NTAB = B * NF
HWPIX = H * W
NP = B * HWPIX

NC, NS, L = 2, 16, 16
NW = NC * NS
PIX_PER_W = NP // NW
C = 512
NG = C // L
IDXB = 128
NIDX = C // IDXB
NCHUNK = PIX_PER_W // C

_MUL = np.uint32(2654435761)


def _hash_pix(pvec_u32):
    hsh = pvec_u32 * _MUL
    tri = (hsh % np.uint32(NTAB)).astype(jnp.int32)
    valid = (hsh % np.uint32(7)) != np.uint32(0)
    validf = jnp.where(valid, np.float32(1.0), np.float32(0.0))
    b0 = ((hsh >> 3) % np.uint32(1024)).astype(jnp.float32) + 1.0
    b1 = ((hsh >> 13) % np.uint32(1024)).astype(jnp.float32) + 1.0
    b2 = ((hsh >> 23) % np.uint32(512)).astype(jnp.float32) + 1.0
    scale = validf / (b0 + b1 + b2)
    return tri, b0 * scale, b1 * scale, b2 * scale


def _body(table, out, idxA, idxB_, wA, wB, gA, gB, tA, tB, gsA, gsB, osA, osB):
    wid = lax.axis_index("s") * NC + lax.axis_index("c")
    pix0 = wid * PIX_PER_W
    b = pix0 // HWPIX
    out0 = (b * 17) * HWPIX + pix0 % HWPIX
    lane = lax.broadcasted_iota(jnp.int32, (16,), 0)
    lane_u = lane.astype(jnp.uint32)

    def gen(c, idxbuf, wbuf):
        pbase = pix0 + c * C

        def one(g, _):
            pvec = (pbase + g * L).astype(jnp.uint32) + lane_u
            tri, w0, w1, w2 = _hash_pix(pvec)
            idxbuf[g // (IDXB // L), pl.ds((g % (IDXB // L)) * L, L)] = tri
            wbuf[0, pl.ds(g * L, L)] = w0
            wbuf[1, pl.ds(g * L, L)] = w1
            wbuf[2, pl.ds(g * L, L)] = w2
            return 0

        lax.fori_loop(0, NG, one, 0, unroll=4)

    def fire_gather(idxbuf, gbuf, gsem):
        for j in range(NIDX):
            pltpu.async_copy(
                table.at[idxbuf.at[j]], gbuf.at[pl.ds(j * IDXB, IDXB)], gsem
            )

    def wait_gather(idxbuf, gbuf, gsem):
        for j in range(NIDX):
            pltpu.make_async_copy(
                table.at[idxbuf.at[j]], gbuf.at[pl.ds(j * IDXB, IDXB)], gsem
            ).wait()

    def wsum(gbuf, wbuf, tbuf):
        def one(g, _):
            rowv = g * L + lane
            w0 = wbuf[0, pl.ds(g * L, L)]
            w1 = wbuf[1, pl.ds(g * L, L)]
            w2 = wbuf[2, pl.ds(g * L, L)]
            accs = []
            for d in range(16):
                r0 = plsc.load_gather(gbuf, [rowv, lane * 0 + d])
                r1 = plsc.load_gather(gbuf, [rowv, lane * 0 + (16 + d)])
                r2 = plsc.load_gather(gbuf, [rowv, lane * 0 + (32 + d)])
                accs.append(w0 * r0 + w1 * r1 + w2 * r2)
            for d in range(16):
                tbuf[d, pl.ds(g * L, L)] = accs[d]
            tbuf[16, pl.ds(g * L, L)] = (w0 + w1) + w2
            return 0

        lax.fori_loop(0, NG, one, 0)

    def fire_out(c, tbuf, osem):
        off = out0 + c * C
        for d in range(17):
            pltpu.async_copy(tbuf.at[d], out.at[pl.ds(off + d * HWPIX, C)], osem)

    def drain_out(tbuf, osem):
        for d in range(17):
            pltpu.make_async_copy(tbuf.at[d], out.at[pl.ds(d * HWPIX, C)], osem).wait()

    gen(0, idxA, wA)
    fire_gather(idxA, gA, gsA)

    def step(t, _):
        c0 = 2 * t
        c1 = c0 + 1
        gen(c1, idxB_, wB)
        fire_gather(idxB_, gB, gsB)

        @pl.when(t > 0)
        def _():
            drain_out(tA, osA)

        wait_gather(idxA, gA, gsA)
        wsum(gA, wA, tA)
        fire_out(c0, tA, osA)

        @pl.when(c0 + 2 < NCHUNK)
        def _():
            gen(c0 + 2, idxA, wA)
            fire_gather(idxA, gA, gsA)

        @pl.when(t > 0)
        def _():
            drain_out(tB, osB)

        wait_gather(idxB_, gB, gsB)
        wsum(gB, wB, tB)
        fire_out(c1, tB, osB)
        return 0

    lax.fori_loop(0, NCHUNK // 2, step, 0)
    drain_out(tA, osA)
    drain_out(tB, osB)


@jax.jit
def _sc_render(table):
    mesh = plsc.VectorSubcoreMesh(core_axis_name="c", subcore_axis_name="s")
    return pl.kernel(
        _body,
        out_type=jax.ShapeDtypeStruct((B * 17 * HWPIX,), jnp.float32),
        mesh=mesh,
        scratch_types=[
            pltpu.VMEM((NIDX, IDXB), jnp.int32),
            pltpu.VMEM((NIDX, IDXB), jnp.int32),
            pltpu.VMEM((3, C), jnp.float32),
            pltpu.VMEM((3, C), jnp.float32),
            pltpu.VMEM((C, 48), jnp.float32),
            pltpu.VMEM((C, 48), jnp.float32),
            pltpu.VMEM((17, C), jnp.float32),
            pltpu.VMEM((17, C), jnp.float32),
            pltpu.SemaphoreType.DMA,
            pltpu.SemaphoreType.DMA,
            pltpu.SemaphoreType.DMA,
            pltpu.SemaphoreType.DMA,
        ],
        compiler_params=pltpu.CompilerParams(
            use_tc_tiling_on_sc=False, needs_layout_passes=False
        ),
    )(table)


def kernel(v, f, attrs):
    del v, f
    table = attrs.reshape(NTAB, 48)
    return _sc_render(table).reshape(B, 17, H, W)


# speedup vs baseline: 44.5970x; 1.0119x over previous
import numpy as np

import jax
import jax.numpy as jnp
from jax import lax
from jax.experimental import pallas as pl
from jax.experimental.pallas import tpu as pltpu
from jax.experimental.pallas import tpu_sc as plsc

H = 512
W = 512
B = 4
NF = 100000
NTAB = B * NF
HWPIX = H * W
NP = B * HWPIX

NC, NS, L = 2, 16, 16
NW = NC * NS
PIX_PER_W = NP // NW
C = 512
NG = C // L
IDXB = 128
NIDX = C // IDXB
NCHUNK = PIX_PER_W // C

_MUL = np.uint32(2654435761)


def _hash_pix(pvec_u32):
    hsh = pvec_u32 * _MUL
    tri = (hsh % np.uint32(NTAB)).astype(jnp.int32)
    valid = (hsh % np.uint32(7)) != np.uint32(0)
    validf = jnp.where(valid, np.float32(1.0), np.float32(0.0))
    b0 = ((hsh >> 3) % np.uint32(1024)).astype(jnp.float32) + 1.0
    b1 = ((hsh >> 13) % np.uint32(1024)).astype(jnp.float32) + 1.0
    b2 = ((hsh >> 23) % np.uint32(512)).astype(jnp.float32) + 1.0
    scale = validf / (b0 + b1 + b2)
    return tri, b0 * scale, b1 * scale, b2 * scale


def _body(table, out, idxA, idxB_, wA, wB, gA, gB, tA, tB, gsA, gsB, osA, osB):
    wid = lax.axis_index("s") * NC + lax.axis_index("c")
    pix0 = wid * PIX_PER_W
    b = pix0 // HWPIX
    row0 = b * 17
    col0 = pix0 % HWPIX
    lane = lax.broadcasted_iota(jnp.int32, (16,), 0)
    lane_u = lane.astype(jnp.uint32)

    def gen(c, idxbuf, wbuf):
        pbase = pix0 + c * C

        def one(g, _):
            pvec = (pbase + g * L).astype(jnp.uint32) + lane_u
            tri, w0, w1, w2 = _hash_pix(pvec)
            idxbuf[g // (IDXB // L), pl.ds((g % (IDXB // L)) * L, L)] = tri
            wbuf[0, pl.ds(g * L, L)] = w0
            wbuf[1, pl.ds(g * L, L)] = w1
            wbuf[2, pl.ds(g * L, L)] = w2
            return 0

        lax.fori_loop(0, NG, one, 0, unroll=4)

    def fire_gather(idxbuf, gbuf, gsem):
        for j in range(NIDX):
            pltpu.async_copy(
                table.at[idxbuf.at[j]], gbuf.at[pl.ds(j * IDXB, IDXB)], gsem
            )

    def wait_gather(idxbuf, gbuf, gsem):
        for j in range(NIDX):
            pltpu.make_async_copy(
                table.at[idxbuf.at[j]], gbuf.at[pl.ds(j * IDXB, IDXB)], gsem
            ).wait()

    def wsum(gbuf, wbuf, tbuf):
        def one(g, _):
            rowv = g * L + lane
            w0 = wbuf[0, pl.ds(g * L, L)]
            w1 = wbuf[1, pl.ds(g * L, L)]
            w2 = wbuf[2, pl.ds(g * L, L)]
            accs = []
            for d in range(16):
                r0 = plsc.load_gather(gbuf, [rowv, lane * 0 + d])
                r1 = plsc.load_gather(gbuf, [rowv, lane * 0 + (16 + d)])
                r2 = plsc.load_gather(gbuf, [rowv, lane * 0 + (32 + d)])
                accs.append(w0 * r0 + w1 * r1 + w2 * r2)
            for d in range(16):
                tbuf[d, pl.ds(g * L, L)] = accs[d]
            tbuf[16, pl.ds(g * L, L)] = (w0 + w1) + w2
            return 0

        lax.fori_loop(0, NG, one, 0)

    def out_slice(c):
        return out.at[pl.ds(row0, 17), pl.ds(col0 + c * C, C)]

    def fire_out(c, tbuf, osem):
        pltpu.async_copy(tbuf, out_slice(c), osem)

    def drain_out(tbuf, osem):
        pltpu.make_async_copy(tbuf, out_slice(0), osem).wait()

    gen(0, idxA, wA)
    fire_gather(idxA, gA, gsA)

    def step(t, _):
        c0 = 2 * t
        c1 = c0 + 1
        gen(c1, idxB_, wB)
        fire_gather(idxB_, gB, gsB)

        @pl.when(t > 0)
        def _():
            drain_out(tA, osA)

        wait_gather(idxA, gA, gsA)
        wsum(gA, wA, tA)
        fire_out(c0, tA, osA)

        @pl.when(c0 + 2 < NCHUNK)
        def _():
            gen(c0 + 2, idxA, wA)
            fire_gather(idxA, gA, gsA)

        @pl.when(t > 0)
        def _():
            drain_out(tB, osB)

        wait_gather(idxB_, gB, gsB)
        wsum(gB, wB, tB)
        fire_out(c1, tB, osB)
        return 0

    lax.fori_loop(0, NCHUNK // 2, step, 0)
    drain_out(tA, osA)
    drain_out(tB, osB)


@jax.jit
def _sc_render(table):
    mesh = plsc.VectorSubcoreMesh(core_axis_name="c", subcore_axis_name="s")
    return pl.kernel(
        _body,
        out_type=jax.ShapeDtypeStruct((B * 17, HWPIX), jnp.float32),
        mesh=mesh,
        scratch_types=[
            pltpu.VMEM((NIDX, IDXB), jnp.int32),
            pltpu.VMEM((NIDX, IDXB), jnp.int32),
            pltpu.VMEM((3, C), jnp.float32),
            pltpu.VMEM((3, C), jnp.float32),
            pltpu.VMEM((C, 48), jnp.float32),
            pltpu.VMEM((C, 48), jnp.float32),
            pltpu.VMEM((17, C), jnp.float32),
            pltpu.VMEM((17, C), jnp.float32),
            pltpu.SemaphoreType.DMA,
            pltpu.SemaphoreType.DMA,
            pltpu.SemaphoreType.DMA,
            pltpu.SemaphoreType.DMA,
        ],
        compiler_params=pltpu.CompilerParams(
            use_tc_tiling_on_sc=False, needs_layout_passes=False
        ),
    )(table)


def kernel(v, f, attrs):
    del v, f
    table = attrs.reshape(NTAB, 48)
    return _sc_render(table).reshape(B, 17, H, W)


# speedup vs baseline: 48.3703x; 1.0846x over previous
import numpy as np

import jax
import jax.numpy as jnp
from jax import lax
from jax.experimental import pallas as pl
from jax.experimental.pallas import tpu as pltpu
from jax.experimental.pallas import tpu_sc as plsc

H = 512
W = 512
B = 4
NF = 100000
NTAB = B * NF
HWPIX = H * W
NP = B * HWPIX

NC, NS, L = 2, 16, 16
NW = NC * NS
PIX_PER_W = NP // NW
C = 512
NG = C // L
IDXB = 128
NIDX = C // IDXB
NCHUNK = PIX_PER_W // C

_MUL = np.uint32(2654435761)


def _hash_pix(pvec_u32):
    hsh = pvec_u32 * _MUL
    tri = (hsh % np.uint32(NTAB)).astype(jnp.int32)
    valid = (hsh % np.uint32(7)) != np.uint32(0)
    validf = jnp.where(valid, np.float32(1.0), np.float32(0.0))
    b0 = ((hsh >> 3) % np.uint32(1024)).astype(jnp.float32) + 1.0
    b1 = ((hsh >> 13) % np.uint32(1024)).astype(jnp.float32) + 1.0
    b2 = ((hsh >> 23) % np.uint32(512)).astype(jnp.float32) + 1.0
    scale = validf / (b0 + b1 + b2)
    return tri, b0 * scale, b1 * scale, b2 * scale


def _body(table, out, idxA, idxB_, wA, wB, gA, gB, tA, tB, gsA, gsB, osA, osB):
    wid = lax.axis_index("s") * NC + lax.axis_index("c")
    pix0 = wid * PIX_PER_W
    b = pix0 // HWPIX
    y0 = (pix0 % HWPIX) // W
    lane = lax.broadcasted_iota(jnp.int32, (16,), 0)
    lane_u = lane.astype(jnp.uint32)

    def gen(c, idxbuf, wbuf):
        pbase = pix0 + c * C

        def one(g, _):
            pvec = (pbase + g * L).astype(jnp.uint32) + lane_u
            tri, w0, w1, w2 = _hash_pix(pvec)
            idxbuf[g // (IDXB // L), pl.ds((g % (IDXB // L)) * L, L)] = tri
            wbuf[0, pl.ds(g * L, L)] = w0
            wbuf[1, pl.ds(g * L, L)] = w1
            wbuf[2, pl.ds(g * L, L)] = w2
            return 0

        lax.fori_loop(0, NG, one, 0, unroll=4)

    def fire_gather(idxbuf, gbuf, gsem):
        for j in range(NIDX):
            pltpu.async_copy(
                table.at[idxbuf.at[j]], gbuf.at[pl.ds(j * IDXB, IDXB)], gsem
            )

    def wait_gather(idxbuf, gbuf, gsem):
        for j in range(NIDX):
            pltpu.make_async_copy(
                table.at[idxbuf.at[j]], gbuf.at[pl.ds(j * IDXB, IDXB)], gsem
            ).wait()

    def wsum(gbuf, wbuf, tbuf):
        def one(g, _):
            rowv = g * L + lane
            w0 = wbuf[0, pl.ds(g * L, L)]
            w1 = wbuf[1, pl.ds(g * L, L)]
            w2 = wbuf[2, pl.ds(g * L, L)]
            accs = []
            for d in range(16):
                r0 = plsc.load_gather(gbuf, [rowv, lane * 0 + d])
                r1 = plsc.load_gather(gbuf, [rowv, lane * 0 + (16 + d)])
                r2 = plsc.load_gather(gbuf, [rowv, lane * 0 + (32 + d)])
                accs.append(w0 * r0 + w1 * r1 + w2 * r2)
            for d in range(16):
                tbuf[d, g // 8, pl.ds((g % 8) * L, L)] = accs[d]
            tbuf[16, g // 8, pl.ds((g % 8) * L, L)] = (w0 + w1) + w2
            return 0

        lax.fori_loop(0, NG, one, 0)

    def out_slice(c):
        y = y0 + c
        return out.at[b, :, y // 8, :, y % 8, :]

    def fire_out(c, tbuf, osem):
        pltpu.async_copy(tbuf, out_slice(c), osem)

    def drain_out(tbuf, osem):
        pltpu.make_async_copy(tbuf, out_slice(0), osem).wait()

    gen(0, idxA, wA)
    fire_gather(idxA, gA, gsA)

    def step(t, _):
        c0 = 2 * t
        c1 = c0 + 1
        gen(c1, idxB_, wB)
        fire_gather(idxB_, gB, gsB)

        @pl.when(t > 0)
        def _():
            drain_out(tA, osA)

        wait_gather(idxA, gA, gsA)
        wsum(gA, wA, tA)
        fire_out(c0, tA, osA)

        @pl.when(c0 + 2 < NCHUNK)
        def _():
            gen(c0 + 2, idxA, wA)
            fire_gather(idxA, gA, gsA)

        @pl.when(t > 0)
        def _():
            drain_out(tB, osB)

        wait_gather(idxB_, gB, gsB)
        wsum(gB, wB, tB)
        fire_out(c1, tB, osB)
        return 0

    lax.fori_loop(0, NCHUNK // 2, step, 0)
    drain_out(tA, osA)
    drain_out(tB, osB)


@jax.jit
def _sc_render(table):
    mesh = plsc.VectorSubcoreMesh(core_axis_name="c", subcore_axis_name="s")
    return pl.kernel(
        _body,
        out_type=jax.ShapeDtypeStruct(
            (B, 17, H // 8, W // 128, 8, 128), jnp.float32
        ),
        mesh=mesh,
        scratch_types=[
            pltpu.VMEM((NIDX, IDXB), jnp.int32),
            pltpu.VMEM((NIDX, IDXB), jnp.int32),
            pltpu.VMEM((3, C), jnp.float32),
            pltpu.VMEM((3, C), jnp.float32),
            pltpu.VMEM((C, 48), jnp.float32),
            pltpu.VMEM((C, 48), jnp.float32),
            pltpu.VMEM((17, W // 128, 128), jnp.float32),
            pltpu.VMEM((17, W // 128, 128), jnp.float32),
            pltpu.SemaphoreType.DMA,
            pltpu.SemaphoreType.DMA,
            pltpu.SemaphoreType.DMA,
            pltpu.SemaphoreType.DMA,
        ],
        compiler_params=pltpu.CompilerParams(
            use_tc_tiling_on_sc=False, needs_layout_passes=False
        ),
    )(table)


def kernel(v, f, attrs):
    del v, f
    table = attrs.reshape(NTAB, 48)
    o6 = _sc_render(table)
    return o6.transpose(0, 1, 2, 4, 3, 5).reshape(B, 17, H, W)
